# baseline (speedup 1.0000x reference)
import jax
import jax.numpy as jnp
from jax.experimental import pallas as pl
from jax.experimental.pallas import tpu as pltpu


def _round_up(x, m):
    return ((x + m - 1) // m) * m


def _fast_exp(s):
    log2e = jnp.float32(1.4426950408889634)
    ln2_hi = jnp.float32(0.6931471824645996)
    ln2_lo = jnp.float32(-1.904654323148236e-09)
    kf = jnp.round(s * log2e)
    r = (s - kf * ln2_hi) - kf * ln2_lo
    p = jnp.float32(1.0 / 24.0)
    p = p * r + jnp.float32(1.0 / 6.0)
    p = p * r + jnp.float32(0.5)
    p = p * r + jnp.float32(1.0)
    p = p * r + jnp.float32(1.0)
    k = kf.astype(jnp.int32)
    scale = jax.lax.bitcast_convert_type((k + 127) << 23, jnp.float32)
    return p * scale


def _mdn_body(x_ref, wsm_ref, bsm_ref, wpi_ref, bpi_ref,
              pi_ref, sigma_ref, mu_ref):
    tb = pi_ref.shape[0]
    i = pl.program_id(0)
    go = sigma_ref.shape[-1]
    ck = 256 if tb % 256 == 0 else tb
    n_chunks = tb // ck

    def chunk(j, _):
        x = x_ref[pl.ds(i * tb + j * ck, ck), :].astype(jnp.bfloat16)
        sm = jnp.dot(x, wsm_ref[...],
                     preferred_element_type=jnp.float32) + bsm_ref[...]
        row = pl.ds(j * ck, ck)
        sigma_ref[row, :] = _fast_exp(sm[:, :go]).astype(sigma_ref.dtype)
        mu_ref[row, :] = sm[:, go:].astype(mu_ref.dtype)
        logits = jnp.dot(x, wpi_ref[...],
                         preferred_element_type=jnp.float32) + bpi_ref[...]
        m = jnp.max(logits, axis=1, keepdims=True)
        e = _fast_exp(logits - m)
        pi_ref[row, :] = (e / jnp.sum(e, axis=1, keepdims=True)
                          ).astype(pi_ref.dtype)
        return 0

    jax.lax.fori_loop(0, n_chunks, chunk, 0)


def kernel(x, w_pi, b_pi, w_sigma, b_sigma, w_mu, b_mu):
    B, D = x.shape
    G = w_pi.shape[1]
    GO = w_sigma.shape[1]
    O = GO // G
    out_dtype = x.dtype

    w_sm = jnp.concatenate([w_sigma, w_mu], axis=1).astype(jnp.bfloat16)
    b_sm = jnp.concatenate([b_sigma, b_mu], axis=1)
    w_pi16 = w_pi.astype(jnp.bfloat16)

    TB = min(2048, max(8, _round_up(-(-B // 4), 8)))
    B_pad = _round_up(B, TB)
    if B_pad != B:
        x = jnp.pad(x, ((0, B_pad - B), (0, 0)))
    grid = (B_pad // TB,)

    pi_pad, sigma_pad, mu_pad = pl.pallas_call(
        _mdn_body,
        out_shape=(
            jax.ShapeDtypeStruct((B_pad, G), out_dtype),
            jax.ShapeDtypeStruct((B_pad, GO), out_dtype),
            jax.ShapeDtypeStruct((B_pad, GO), out_dtype),
        ),
        grid=grid,
        in_specs=[
            pl.BlockSpec((B_pad, D), lambda i: (0, 0)),
            pl.BlockSpec((D, 2 * GO), lambda i: (0, 0)),
            pl.BlockSpec((1, 2 * GO), lambda i: (0, 0)),
            pl.BlockSpec((D, G), lambda i: (0, 0)),
            pl.BlockSpec((1, G), lambda i: (0, 0)),
        ],
        out_specs=(
            pl.BlockSpec((TB, G), lambda i: (i, 0)),
            pl.BlockSpec((TB, GO), lambda i: (i, 0)),
            pl.BlockSpec((TB, GO), lambda i: (i, 0)),
        ),
        compiler_params=pltpu.CompilerParams(
            dimension_semantics=("arbitrary",),
            vmem_limit_bytes=64 * 1024 * 1024,
        ),
    )(x, w_sm, b_sm, w_pi16, b_pi)

    if B_pad != B:
        pi_pad = pi_pad[:B]
        sigma_pad = sigma_pad[:B]
        mu_pad = mu_pad[:B]
    return pi_pad, sigma_pad.reshape(B, G, O), mu_pad.reshape(B, G, O)


# speedup vs baseline: 1.2587x; 1.2587x over previous
import jax
import jax.numpy as jnp
from jax.experimental import pallas as pl
from jax.experimental.pallas import tpu as pltpu


def _round_up(x, m):
    return ((x + m - 1) // m) * m


def _fast_exp(s):
    log2e = jnp.float32(1.4426950408889634)
    ln2_hi = jnp.float32(0.6931471824645996)
    ln2_lo = jnp.float32(-1.904654323148236e-09)
    kf = jnp.round(s * log2e)
    r = (s - kf * ln2_hi) - kf * ln2_lo
    p = jnp.float32(1.0 / 24.0)
    p = p * r + jnp.float32(1.0 / 6.0)
    p = p * r + jnp.float32(0.5)
    p = p * r + jnp.float32(1.0)
    p = p * r + jnp.float32(1.0)
    k = kf.astype(jnp.int32)
    scale = jax.lax.bitcast_convert_type((k + 127) << 23, jnp.float32)
    return p * scale


def _hybrid_exp(s, eup_rows):
    if eup_rows <= 0:
        return _fast_exp(s)
    if eup_rows >= s.shape[0]:
        return jnp.exp(s)
    return jnp.concatenate(
        [jnp.exp(s[:eup_rows, :]), _fast_exp(s[eup_rows:, :])], axis=0)


def _make_body(n_steps, eup_rows):
    def body(x_ref, wsm_ref, bsm_ref, wpi_ref, bpi_ref,
             pi_ref, sigma_ref, mu_ref,
             pi_s, sigma_s, mu_s):
        i = pl.program_id(0)
        tb = pi_ref.shape[0]
        go = sigma_ref.shape[-1]

        @pl.when(i < n_steps)
        def _compute():
            x = x_ref[...].astype(jnp.bfloat16)
            sm = jnp.dot(x, wsm_ref[...],
                         preferred_element_type=jnp.float32) + bsm_ref[...]
            row = pl.ds(i * tb, tb)
            sigma_s[row, :] = _hybrid_exp(sm[:, :go], eup_rows)
            mu_s[row, :] = sm[:, go:]
            logits = jnp.dot(x, wpi_ref[...],
                             preferred_element_type=jnp.float32) + bpi_ref[...]
            m = jnp.max(logits, axis=1, keepdims=True)
            e = jnp.exp(logits - m)
            pi_s[row, :] = e / jnp.sum(e, axis=1, keepdims=True)

        @pl.when(i >= n_steps)
        def _writeback():
            row = pl.ds((i - n_steps) * tb, tb)
            pi_ref[...] = pi_s[row, :]
            sigma_ref[...] = sigma_s[row, :]
            mu_ref[...] = mu_s[row, :]

    return body


def kernel(x, w_pi, b_pi, w_sigma, b_sigma, w_mu, b_mu):
    B, D = x.shape
    G = w_pi.shape[1]
    GO = w_sigma.shape[1]
    O = GO // G
    out_dtype = x.dtype

    w_sm = jnp.concatenate([w_sigma, w_mu], axis=1).astype(jnp.bfloat16)
    b_sm = jnp.concatenate([b_sigma, b_mu], axis=1)
    w_pi16 = w_pi.astype(jnp.bfloat16)

    TB = min(2048, max(8, _round_up(-(-B // 4), 8)))
    B_pad = _round_up(B, TB)
    if B_pad != B:
        x = jnp.pad(x, ((0, B_pad - B), (0, 0)))
    n = B_pad // TB
    grid = (2 * n,)
    last = n - 1
    eup_rows = _round_up(TB // 4, 8)

    pi_pad, sigma_pad, mu_pad = pl.pallas_call(
        _make_body(n, eup_rows),
        out_shape=(
            jax.ShapeDtypeStruct((B_pad, G), out_dtype),
            jax.ShapeDtypeStruct((B_pad, GO), out_dtype),
            jax.ShapeDtypeStruct((B_pad, GO), out_dtype),
        ),
        grid=grid,
        in_specs=[
            pl.BlockSpec((TB, D), lambda i: (jnp.minimum(i, last), 0)),
            pl.BlockSpec((D, 2 * GO), lambda i: (0, 0)),
            pl.BlockSpec((1, 2 * GO), lambda i: (0, 0)),
            pl.BlockSpec((D, G), lambda i: (0, 0)),
            pl.BlockSpec((1, G), lambda i: (0, 0)),
        ],
        out_specs=(
            pl.BlockSpec((TB, G), lambda i: (jnp.maximum(i - n, 0), 0)),
            pl.BlockSpec((TB, GO), lambda i: (jnp.maximum(i - n, 0), 0)),
            pl.BlockSpec((TB, GO), lambda i: (jnp.maximum(i - n, 0), 0)),
        ),
        scratch_shapes=[
            pltpu.VMEM((B_pad, G), jnp.float32),
            pltpu.VMEM((B_pad, GO), jnp.float32),
            pltpu.VMEM((B_pad, GO), jnp.float32),
        ],
        compiler_params=pltpu.CompilerParams(
            dimension_semantics=("arbitrary",),
            vmem_limit_bytes=64 * 1024 * 1024,
        ),
    )(x, w_sm, b_sm, w_pi16, b_pi)

    if B_pad != B:
        pi_pad = pi_pad[:B]
        sigma_pad = sigma_pad[:B]
        mu_pad = mu_pad[:B]
    return pi_pad, sigma_pad.reshape(B, G, O), mu_pad.reshape(B, G, O)


# speedup vs baseline: 1.2712x; 1.0099x over previous
import jax
import jax.numpy as jnp
from jax.experimental import pallas as pl
from jax.experimental.pallas import tpu as pltpu


def _round_up(x, m):
    return ((x + m - 1) // m) * m


def _fast_exp(s):
    log2e = jnp.float32(1.4426950408889634)
    ln2_hi = jnp.float32(0.6931471824645996)
    ln2_lo = jnp.float32(-1.904654323148236e-09)
    kf = jnp.round(s * log2e)
    r = (s - kf * ln2_hi) - kf * ln2_lo
    p = jnp.float32(1.0 / 24.0)
    p = p * r + jnp.float32(1.0 / 6.0)
    p = p * r + jnp.float32(0.5)
    p = p * r + jnp.float32(1.0)
    p = p * r + jnp.float32(1.0)
    k = kf.astype(jnp.int32)
    scale = jax.lax.bitcast_convert_type((k + 127) << 23, jnp.float32)
    return p * scale


def _make_body(eup_rows):
    def body(x_ref, wsm_ref, bsm_ref, wpi_ref, bpi_ref,
             pi_ref, sigma_ref, mu_ref):
        tb = pi_ref.shape[0]
        i = pl.program_id(0)
        go = sigma_ref.shape[-1]
        x = x_ref[pl.ds(i * tb, tb), :].astype(jnp.bfloat16)

        sm = jnp.dot(x, wsm_ref[...],
                     preferred_element_type=jnp.float32) + bsm_ref[...]
        s = sm[:, :go]
        if 0 < eup_rows < tb:
            sigma_ref[:eup_rows, :] = jnp.exp(s[:eup_rows, :])
            sigma_ref[eup_rows:, :] = _fast_exp(s[eup_rows:, :])
        else:
            sigma_ref[...] = _fast_exp(s)
        mu_ref[...] = sm[:, go:]

        logits = jnp.dot(x, wpi_ref[...],
                         preferred_element_type=jnp.float32) + bpi_ref[...]
        m = jnp.max(logits, axis=1, keepdims=True)
        e = _fast_exp(logits - m)
        pi_ref[...] = e / jnp.sum(e, axis=1, keepdims=True)

    return body


def kernel(x, w_pi, b_pi, w_sigma, b_sigma, w_mu, b_mu):
    B, D = x.shape
    G = w_pi.shape[1]
    GO = w_sigma.shape[1]
    O = GO // G
    out_dtype = x.dtype

    w_sm = jnp.concatenate([w_sigma, w_mu], axis=1).astype(jnp.bfloat16)
    b_sm = jnp.concatenate([b_sigma, b_mu], axis=1)
    w_pi16 = w_pi.astype(jnp.bfloat16)

    TB = min(2048, max(8, _round_up(-(-B // 4), 8)))
    B_pad = _round_up(B, TB)
    if B_pad != B:
        x = jnp.pad(x, ((0, B_pad - B), (0, 0)))
    grid = (B_pad // TB,)
    eup_rows = _round_up(TB // 4, 8) if TB >= 32 else 0

    pi_pad, sigma_pad, mu_pad = pl.pallas_call(
        _make_body(eup_rows),
        out_shape=(
            jax.ShapeDtypeStruct((B_pad, G), out_dtype),
            jax.ShapeDtypeStruct((B_pad, GO), out_dtype),
            jax.ShapeDtypeStruct((B_pad, GO), out_dtype),
        ),
        grid=grid,
        in_specs=[
            pl.BlockSpec((B_pad, D), lambda i: (0, 0)),
            pl.BlockSpec((D, 2 * GO), lambda i: (0, 0)),
            pl.BlockSpec((1, 2 * GO), lambda i: (0, 0)),
            pl.BlockSpec((D, G), lambda i: (0, 0)),
            pl.BlockSpec((1, G), lambda i: (0, 0)),
        ],
        out_specs=(
            pl.BlockSpec((TB, G), lambda i: (i, 0)),
            pl.BlockSpec((TB, GO), lambda i: (i, 0)),
            pl.BlockSpec((TB, GO), lambda i: (i, 0)),
        ),
        compiler_params=pltpu.CompilerParams(
            dimension_semantics=("arbitrary",),
            vmem_limit_bytes=64 * 1024 * 1024,
        ),
    )(x, w_sm, b_sm, w_pi16, b_pi)

    if B_pad != B:
        pi_pad = pi_pad[:B]
        sigma_pad = sigma_pad[:B]
        mu_pad = mu_pad[:B]
    return pi_pad, sigma_pad.reshape(B, G, O), mu_pad.reshape(B, G, O)


# speedup vs baseline: 1.3827x; 1.0877x over previous
import jax
import jax.numpy as jnp
from jax.experimental import pallas as pl
from jax.experimental.pallas import tpu as pltpu


def _round_up(x, m):
    return ((x + m - 1) // m) * m


def _fast_exp(s):
    log2e = jnp.float32(1.4426950408889634)
    ln2_hi = jnp.float32(0.6931471824645996)
    ln2_lo = jnp.float32(-1.904654323148236e-09)
    kf = jnp.round(s * log2e)
    r = (s - kf * ln2_hi) - kf * ln2_lo
    p = jnp.float32(1.0 / 24.0)
    p = p * r + jnp.float32(1.0 / 6.0)
    p = p * r + jnp.float32(0.5)
    p = p * r + jnp.float32(1.0)
    p = p * r + jnp.float32(1.0)
    k = kf.astype(jnp.int32)
    scale = jax.lax.bitcast_convert_type((k + 127) << 23, jnp.float32)
    return p * scale


def _mdn_body(x_ref, wsm_ref, bsm_ref, wpi_ref, bpiT_ref,
              pi_ref, sigma_ref, mu_ref):
    tb = pi_ref.shape[0]
    i = pl.program_id(0)
    go = sigma_ref.shape[-1]
    x = x_ref[pl.ds(i * tb, tb), :].astype(jnp.bfloat16)

    sm = jnp.dot(x, wsm_ref[...],
                 preferred_element_type=jnp.float32) + bsm_ref[...]
    sigma_ref[...] = jnp.exp(sm[:, :go])
    mu_ref[...] = sm[:, go:]

    logitsT = jax.lax.dot_general(
        wpi_ref[...], x, (((0,), (1,)), ((), ())),
        preferred_element_type=jnp.float32) + bpiT_ref[...]
    m = jnp.max(logitsT, axis=0, keepdims=True)
    e = _fast_exp(logitsT - m)
    piT = e / jnp.sum(e, axis=0, keepdims=True)
    pi_ref[...] = piT.T


def kernel(x, w_pi, b_pi, w_sigma, b_sigma, w_mu, b_mu):
    B, D = x.shape
    G = w_pi.shape[1]
    GO = w_sigma.shape[1]
    O = GO // G
    out_dtype = x.dtype

    w_sm = jnp.concatenate([w_sigma, w_mu], axis=1).astype(jnp.bfloat16)
    b_sm = jnp.concatenate([b_sigma, b_mu], axis=1)
    w_pi16 = w_pi.astype(jnp.bfloat16)
    b_piT = b_pi.reshape(G, 1)

    TB = min(2048, max(8, _round_up(-(-B // 4), 8)))
    B_pad = _round_up(B, TB)
    if B_pad != B:
        x = jnp.pad(x, ((0, B_pad - B), (0, 0)))
    grid = (B_pad // TB,)

    pi_pad, sigma_pad, mu_pad = pl.pallas_call(
        _mdn_body,
        out_shape=(
            jax.ShapeDtypeStruct((B_pad, G), out_dtype),
            jax.ShapeDtypeStruct((B_pad, GO), out_dtype),
            jax.ShapeDtypeStruct((B_pad, GO), out_dtype),
        ),
        grid=grid,
        in_specs=[
            pl.BlockSpec((B_pad, D), lambda i: (0, 0)),
            pl.BlockSpec((D, 2 * GO), lambda i: (0, 0)),
            pl.BlockSpec((1, 2 * GO), lambda i: (0, 0)),
            pl.BlockSpec((D, G), lambda i: (0, 0)),
            pl.BlockSpec((G, 1), lambda i: (0, 0)),
        ],
        out_specs=(
            pl.BlockSpec((TB, G), lambda i: (i, 0)),
            pl.BlockSpec((TB, GO), lambda i: (i, 0)),
            pl.BlockSpec((TB, GO), lambda i: (i, 0)),
        ),
        compiler_params=pltpu.CompilerParams(
            dimension_semantics=("arbitrary",),
            vmem_limit_bytes=64 * 1024 * 1024,
        ),
    )(x, w_sm, b_sm, w_pi16, b_piT)

    if B_pad != B:
        pi_pad = pi_pad[:B]
        sigma_pad = sigma_pad[:B]
        mu_pad = mu_pad[:B]
    return pi_pad, sigma_pad.reshape(B, G, O), mu_pad.reshape(B, G, O)
